# baseline (speedup 1.0000x reference)
import jax
import jax.numpy as jnp
from jax import lax
from jax.experimental import pallas as pl
from jax.experimental.pallas import tpu as pltpu
from jax.experimental.pallas import tpu_sc as plsc

_SDF_MIN = -7.0
_SDF_MAX = 7.0
_N_BINS = 256

_NC = 2
_NS = 16
_NW = _NC * _NS
_L = 16

_N = 8 * 128 * 128 * 128
_CHUNK = 16384
_UNROLL = 8

_SC_UNIT = _NW * _CHUNK
_SC_UNITS = 22
_F = _SC_UNITS * _SC_UNIT
_PER_W = _F // _NW
_NCHUNKS = _PER_W // _CHUNK

_R = _N - _F

_A = (_N_BINS - 1) / (_SDF_MAX - _SDF_MIN)
_B = -_SDF_MIN * _A + 0.5


def _compute_chunk(yp_v, yt_v, lut_v, accs):
    @plsc.parallel_loop(0, _CHUNK, step=_UNROLL * _L, carry=accs)
    def body(off, acc_in):
        acc = list(acc_in)
        for j in range(_UNROLL):
            t = yt_v[pl.ds(off + j * _L, _L)]
            p = yp_v[pl.ds(off + j * _L, _L)]
            tc = jnp.minimum(jnp.maximum(t, _SDF_MIN), _SDF_MAX)
            x = tc * _A + _B
            idx = x.astype(jnp.int32)
            w = plsc.load_gather(lut_v, [idx])
            d = p - t
            acc[j] = acc[j] + w * (d * d)
        return tuple(acc)

    return body


def _sc_body(yp_hbm, yt_hbm, lut_hbm, out_hbm,
             lut_v, yp0, yp1, yt0, yt1, acc_v,
             sp0, sp1, st0, st1):
    c = lax.axis_index("c")
    s = lax.axis_index("s")
    wid = s * _NC + c
    base = wid * _PER_W
    pltpu.sync_copy(lut_hbm, lut_v)

    bufs = ((yp0, yt0, sp0, st0), (yp1, yt1, sp1, st1))

    def start(k, parity):
        ypb, ytb, sp, st = bufs[parity]
        off = base + k * _CHUNK
        pltpu.async_copy(yp_hbm.at[pl.ds(off, _CHUNK)], ypb, sp)
        pltpu.async_copy(yt_hbm.at[pl.ds(off, _CHUNK)], ytb, st)

    def wait(parity):
        ypb, ytb, sp, st = bufs[parity]
        pltpu.make_async_copy(yp_hbm.at[pl.ds(base, _CHUNK)], ypb, sp).wait()
        pltpu.make_async_copy(yt_hbm.at[pl.ds(base, _CHUNK)], ytb, st).wait()

    start(0, 0)
    start(1, 1)

    zeros = jnp.zeros((_L,), jnp.float32)
    accs0 = (zeros,) * _UNROLL

    def pair_body(g, accs):
        k0 = 2 * g
        wait(0)
        accs = _compute_chunk(bufs[0][0], bufs[0][1], lut_v, accs)
        start(k0 + 2, 0)
        wait(1)
        accs = _compute_chunk(bufs[1][0], bufs[1][1], lut_v, accs)
        start(k0 + 3, 1)
        return accs

    accs = lax.fori_loop(0, _NCHUNKS // 2 - 1, pair_body, accs0)

    wait(0)
    accs = _compute_chunk(bufs[0][0], bufs[0][1], lut_v, accs)
    wait(1)
    accs = _compute_chunk(bufs[1][0], bufs[1][1], lut_v, accs)

    half = len(accs) // 2
    acc = sum(accs[1:half], accs[0]) + sum(accs[half + 1:], accs[half])
    acc_v[...] = acc
    pltpu.sync_copy(acc_v, out_hbm.at[wid])


def _sc_partials(yp, yt, lut):
    mesh = plsc.VectorSubcoreMesh(core_axis_name="c", subcore_axis_name="s")
    return pl.kernel(
        _sc_body,
        out_type=jax.ShapeDtypeStruct((_NW, _L), jnp.float32),
        mesh=mesh,
        scratch_types=[
            pltpu.VMEM((_N_BINS,), jnp.float32),
            pltpu.VMEM((_CHUNK,), jnp.float32),
            pltpu.VMEM((_CHUNK,), jnp.float32),
            pltpu.VMEM((_CHUNK,), jnp.float32),
            pltpu.VMEM((_CHUNK,), jnp.float32),
            pltpu.VMEM((_L,), jnp.float32),
            pltpu.SemaphoreType.DMA,
            pltpu.SemaphoreType.DMA,
            pltpu.SemaphoreType.DMA,
            pltpu.SemaphoreType.DMA,
        ],
        compiler_params=pltpu.CompilerParams(needs_layout_passes=False),
    )(yp, yt, lut)


def _tc_body(yp_ref, yt_ref, lut_ref, out_ref):
    i = pl.program_id(0)
    yp = yp_ref[...]
    yt = yt_ref[...]
    tc = jnp.minimum(jnp.maximum(yt, _SDF_MIN), _SDF_MAX)
    x = tc * _A + _B
    idx = x.astype(jnp.int32)
    lo = jnp.broadcast_to(lut_ref[0:1, 0:128], (8, 128))
    hi = jnp.broadcast_to(lut_ref[0:1, 128:256], (8, 128))
    idxm = idx & 127
    wlo = jnp.take_along_axis(lo, idxm, axis=1)
    whi = jnp.take_along_axis(hi, idxm, axis=1)
    w = jnp.where(idx < 128, wlo, whi)
    d = yp - yt
    s = jnp.sum(w * (d * d))

    @pl.when(i == 0)
    def _():
        out_ref[...] = jnp.zeros((1, 1), jnp.float32)

    out_ref[...] += jnp.full((1, 1), 1.0, jnp.float32) * s


def _tc_total(yp, yt, lut):
    cols = _R // 8
    grid = 5
    bc = cols // grid
    total = pl.pallas_call(
        _tc_body,
        grid=(grid,),
        in_specs=[
            pl.BlockSpec((8, bc), lambda i: (0, i)),
            pl.BlockSpec((8, bc), lambda i: (0, i)),
            pl.BlockSpec((1, _N_BINS), lambda i: (0, 0)),
        ],
        out_specs=pl.BlockSpec((1, 1), lambda i: (0, 0)),
        out_shape=jax.ShapeDtypeStruct((1, 1), jnp.float32),
    )(yp.reshape(8, cols), yt.reshape(8, cols), lut.reshape(1, _N_BINS))
    return total[0, 0]


def kernel(y_pred, y_true, lut):
    n = y_pred.size
    ypf = y_pred.reshape(-1)
    ytf = y_true.reshape(-1)
    partials = _sc_partials(ypf[:_F], ytf[:_F], lut)
    tc_sum = _tc_total(ypf[_F:], ytf[_F:], lut)
    return ((partials.sum() + tc_sum) / n).astype(jnp.float32)


# speedup vs baseline: 5.2595x; 5.2595x over previous
import jax
import jax.numpy as jnp
from jax import lax
from jax.experimental import pallas as pl
from jax.experimental.pallas import tpu as pltpu
from jax.experimental.pallas import tpu_sc as plsc

_SDF_MIN = -7.0
_SDF_MAX = 7.0
_N_BINS = 256

_NC = 2
_NS = 16
_NW = _NC * _NS
_L = 16

_N = 8 * 128 * 128 * 128
_CHUNK = 16384
_UNROLL = 8

_SC_UNIT = _NW * _CHUNK
_SC_UNITS = 22
_F = _SC_UNITS * _SC_UNIT
_PER_W = _F // _NW
_NCHUNKS = _PER_W // _CHUNK

_R = _N - _F

_A = (_N_BINS - 1) / (_SDF_MAX - _SDF_MIN)
_B = -_SDF_MIN * _A + 0.5


def _compute_chunk(yp_v, yt_v, lut_v, accs):
    @plsc.parallel_loop(0, _CHUNK, step=_UNROLL * _L, carry=accs)
    def body(off, acc_in):
        acc = list(acc_in)
        for j in range(_UNROLL):
            t = yt_v[pl.ds(off + j * _L, _L)]
            p = yp_v[pl.ds(off + j * _L, _L)]
            tc = jnp.minimum(jnp.maximum(t, _SDF_MIN), _SDF_MAX)
            x = tc * _A + _B
            idx = x.astype(jnp.int32)
            w = plsc.load_gather(lut_v, [idx])
            d = p - t
            acc[j] = acc[j] + w * (d * d)
        return tuple(acc)

    return body


def _sc_body(yp_hbm, yt_hbm, lut_hbm, out_hbm,
             lut_v, yp0, yp1, yt0, yt1, acc_v,
             sp0, sp1, st0, st1):
    c = lax.axis_index("c")
    s = lax.axis_index("s")
    wid = s * _NC + c
    base = wid * _PER_W
    pltpu.sync_copy(lut_hbm, lut_v)

    bufs = ((yp0, yt0, sp0, st0), (yp1, yt1, sp1, st1))

    def start(k, parity):
        ypb, ytb, sp, st = bufs[parity]
        off = base + k * _CHUNK
        pltpu.async_copy(yp_hbm.at[pl.ds(off, _CHUNK)], ypb, sp)
        pltpu.async_copy(yt_hbm.at[pl.ds(off, _CHUNK)], ytb, st)

    def wait(parity):
        ypb, ytb, sp, st = bufs[parity]
        pltpu.make_async_copy(yp_hbm.at[pl.ds(base, _CHUNK)], ypb, sp).wait()
        pltpu.make_async_copy(yt_hbm.at[pl.ds(base, _CHUNK)], ytb, st).wait()

    start(0, 0)
    start(1, 1)

    zeros = jnp.zeros((_L,), jnp.float32)
    accs0 = (zeros,) * _UNROLL

    def pair_body(g, accs):
        k0 = 2 * g
        wait(0)
        accs = _compute_chunk(bufs[0][0], bufs[0][1], lut_v, accs)
        start(k0 + 2, 0)
        wait(1)
        accs = _compute_chunk(bufs[1][0], bufs[1][1], lut_v, accs)
        start(k0 + 3, 1)
        return accs

    accs = lax.fori_loop(0, _NCHUNKS // 2 - 1, pair_body, accs0)

    wait(0)
    accs = _compute_chunk(bufs[0][0], bufs[0][1], lut_v, accs)
    wait(1)
    accs = _compute_chunk(bufs[1][0], bufs[1][1], lut_v, accs)

    half = len(accs) // 2
    acc = sum(accs[1:half], accs[0]) + sum(accs[half + 1:], accs[half])
    acc_v[...] = acc
    pltpu.sync_copy(acc_v, out_hbm.at[wid])


def _sc_partials(yp, yt, lut):
    mesh = plsc.VectorSubcoreMesh(core_axis_name="c", subcore_axis_name="s")
    return pl.kernel(
        _sc_body,
        out_type=jax.ShapeDtypeStruct((_NW, _L), jnp.float32),
        mesh=mesh,
        scratch_types=[
            pltpu.VMEM((_N_BINS,), jnp.float32),
            pltpu.VMEM((_CHUNK,), jnp.float32),
            pltpu.VMEM((_CHUNK,), jnp.float32),
            pltpu.VMEM((_CHUNK,), jnp.float32),
            pltpu.VMEM((_CHUNK,), jnp.float32),
            pltpu.VMEM((_L,), jnp.float32),
            pltpu.SemaphoreType.DMA,
            pltpu.SemaphoreType.DMA,
            pltpu.SemaphoreType.DMA,
            pltpu.SemaphoreType.DMA,
        ],
        compiler_params=pltpu.CompilerParams(needs_layout_passes=False),
    )(yp, yt, lut)


def _tc_body(yp_ref, yt_ref, lut_ref, out_ref):
    i = pl.program_id(0)
    yp = yp_ref[...]
    yt = yt_ref[...]
    tc = jnp.minimum(jnp.maximum(yt, _SDF_MIN), _SDF_MAX)
    x = tc * _A + _B
    idx = x.astype(jnp.int32)
    br = idx.shape[0]
    lo = jnp.broadcast_to(lut_ref[0:1, 0:128], (br, 128))
    hi = jnp.broadcast_to(lut_ref[0:1, 128:256], (br, 128))
    idxm = idx & 127
    wlo = jnp.take_along_axis(lo, idxm, axis=1)
    whi = jnp.take_along_axis(hi, idxm, axis=1)
    w = jnp.where(idx < 128, wlo, whi)
    d = yp - yt
    s = jnp.sum(w * (d * d))

    @pl.when(i == 0)
    def _():
        out_ref[...] = jnp.zeros((1, 1), jnp.float32)

    out_ref[...] += jnp.full((1, 1), 1.0, jnp.float32) * s


_ROWS = 8192
_COLS = _N // _ROWS
_SC_ROWS = _F // _COLS
_TC_ROWS = _ROWS - _SC_ROWS
_TC_GRID = 5
_TC_BR = _TC_ROWS // _TC_GRID


def _tc_total(yp, yt, lut):
    start = _SC_ROWS // _TC_BR
    total = pl.pallas_call(
        _tc_body,
        grid=(_TC_GRID,),
        in_specs=[
            pl.BlockSpec((_TC_BR, _COLS), lambda i: (start + i, 0)),
            pl.BlockSpec((_TC_BR, _COLS), lambda i: (start + i, 0)),
            pl.BlockSpec((1, _N_BINS), lambda i: (0, 0)),
        ],
        out_specs=pl.BlockSpec((1, 1), lambda i: (0, 0)),
        out_shape=jax.ShapeDtypeStruct((1, 1), jnp.float32),
    )(yp, yt, lut.reshape(1, _N_BINS))
    return total[0, 0]


def kernel(y_pred, y_true, lut):
    n = y_pred.size
    partials = _sc_partials(y_pred.reshape(-1), y_true.reshape(-1), lut)
    tc_sum = _tc_total(y_pred.reshape(_ROWS, _COLS),
                       y_true.reshape(_ROWS, _COLS), lut)
    return ((partials.sum() + tc_sum) / n).astype(jnp.float32)


# speedup vs baseline: 10.6539x; 2.0257x over previous
import functools

import jax
import jax.numpy as jnp
from jax import lax
from jax.experimental import pallas as pl
from jax.experimental.pallas import tpu as pltpu
from jax.experimental.pallas import tpu_sc as plsc

_SDF_MIN = -7.0
_SDF_MAX = 7.0
_N_BINS = 256

_NC = 2
_NS = 16
_NW = _NC * _NS
_L = 16

_N = 8 * 128 * 128 * 128
_PER_W = _N // _NW
_CHUNK = 16384
_NCHUNKS = _PER_W // _CHUNK
_UNROLL = 8

_A = (_N_BINS - 1) / (_SDF_MAX - _SDF_MIN)
_B = -_SDF_MIN * _A + 0.5


def _compute_chunk(yp_v, yt_v, lut_v, accs):
    def body(i, acc_in):
        off = i * (_UNROLL * _L)
        acc = list(acc_in)
        for j in range(_UNROLL):
            t = yt_v[pl.ds(off + j * _L, _L)]
            p = yp_v[pl.ds(off + j * _L, _L)]
            tc = jnp.minimum(jnp.maximum(t, _SDF_MIN), _SDF_MAX)
            x = tc * _A + _B
            idx = x.astype(jnp.int32)
            w = plsc.load_gather(lut_v, [idx])
            d = p - t
            acc[j] = acc[j] + w * (d * d)
        return tuple(acc)

    return lax.fori_loop(0, _CHUNK // (_UNROLL * _L), body, accs)


def _sc_body(yp_hbm, yt_hbm, lut_hbm, out_hbm,
             lut_v, yp0, yp1, yt0, yt1, acc_v,
             sp0, sp1, st0, st1):
    c = lax.axis_index("c")
    s = lax.axis_index("s")
    wid = s * _NC + c
    base = wid * _PER_W
    pltpu.sync_copy(lut_hbm, lut_v)

    bufs = ((yp0, yt0, sp0, st0), (yp1, yt1, sp1, st1))

    def start(k, parity):
        ypb, ytb, sp, st = bufs[parity]
        off = base + k * _CHUNK
        pltpu.async_copy(yp_hbm.at[pl.ds(off, _CHUNK)], ypb, sp)
        pltpu.async_copy(yt_hbm.at[pl.ds(off, _CHUNK)], ytb, st)

    def wait(parity):
        ypb, ytb, sp, st = bufs[parity]
        pltpu.make_async_copy(yp_hbm.at[pl.ds(base, _CHUNK)], ypb, sp).wait()
        pltpu.make_async_copy(yt_hbm.at[pl.ds(base, _CHUNK)], ytb, st).wait()

    start(0, 0)
    start(1, 1)

    zeros = jnp.zeros((_L,), jnp.float32)
    accs0 = (zeros,) * _UNROLL

    def pair_body(g, accs):
        k0 = 2 * g
        wait(0)
        accs = _compute_chunk(bufs[0][0], bufs[0][1], lut_v, accs)
        start(k0 + 2, 0)
        wait(1)
        accs = _compute_chunk(bufs[1][0], bufs[1][1], lut_v, accs)
        start(k0 + 3, 1)
        return accs

    accs = lax.fori_loop(0, _NCHUNKS // 2 - 1, pair_body, accs0)

    wait(0)
    accs = _compute_chunk(bufs[0][0], bufs[0][1], lut_v, accs)
    wait(1)
    accs = _compute_chunk(bufs[1][0], bufs[1][1], lut_v, accs)

    half = len(accs) // 2
    acc = sum(accs[1:half], accs[0]) + sum(accs[half + 1:], accs[half])
    acc_v[...] = acc
    pltpu.sync_copy(acc_v, out_hbm.at[wid])


@jax.jit
def _sc_partials(yp, yt, lut):
    mesh = plsc.VectorSubcoreMesh(core_axis_name="c", subcore_axis_name="s")
    return pl.kernel(
        _sc_body,
        out_type=jax.ShapeDtypeStruct((_NW, _L), jnp.float32),
        mesh=mesh,
        scratch_types=[
            pltpu.VMEM((_N_BINS,), jnp.float32),
            pltpu.VMEM((_CHUNK,), jnp.float32),
            pltpu.VMEM((_CHUNK,), jnp.float32),
            pltpu.VMEM((_CHUNK,), jnp.float32),
            pltpu.VMEM((_CHUNK,), jnp.float32),
            pltpu.VMEM((_L,), jnp.float32),
            pltpu.SemaphoreType.DMA,
            pltpu.SemaphoreType.DMA,
            pltpu.SemaphoreType.DMA,
            pltpu.SemaphoreType.DMA,
        ],
        compiler_params=pltpu.CompilerParams(needs_layout_passes=False),
    )(yp, yt, lut)


def kernel(y_pred, y_true, lut):
    n = y_pred.size
    partials = _sc_partials(y_pred.reshape(-1), y_true.reshape(-1), lut)
    return (partials.sum() / n).astype(jnp.float32)
